# baseline (speedup 1.0000x reference)
import functools

import jax
import jax.numpy as jnp
from jax import lax
from jax.experimental import pallas as pl
from jax.experimental.pallas import tpu as pltpu
from jax.experimental.pallas import tpu_sc as plsc

EMB_DIM = 32
NUM_CORES = 2
NUM_SUBCORES = 16
G = 4
TOK = G * 128
SB_PER_S = 128 // G


@functools.lru_cache(maxsize=None)
def _make_gather(n_seq, seq_len):
    NW = NUM_CORES * NUM_SUBCORES
    B = n_seq * seq_len
    assert B % (NW * TOK) == 0
    n = B // (NW * TOK)
    n_tiles = n_seq // 128
    s_tiles = seq_len // 8
    mesh = plsc.VectorSubcoreMesh(core_axis_name="c", subcore_axis_name="s")

    scratch = (
        [pltpu.VMEM((G, 128), jnp.int32)] * 2
        + [pltpu.VMEM((TOK, EMB_DIM), jnp.float32)] * 2
        + [pltpu.VMEM((4, G, 8, 128), jnp.float32)] * 2
        + [pltpu.SemaphoreType.DMA] * 6
    )

    @functools.partial(
        pl.kernel,
        out_type=jax.ShapeDtypeStruct(
            (seq_len, 4, n_tiles, 8, 128), jnp.float32
        ),
        mesh=mesh,
        scratch_types=scratch,
        compiler_params=pltpu.CompilerParams(
            use_tc_tiling_on_sc=False, needs_layout_passes=False
        ),
    )
    def gather_kernel(idxp_hbm, table_hbm, out_hbm, *scr):
        idxv = scr[0:2]
        rowsv = scr[2:4]
        mv = scr[4:6]
        isem = scr[6:8]
        gsem = scr[8:10]
        ssem = scr[10:12]

        wid = lax.axis_index("s") * NUM_CORES + lax.axis_index("c")
        u0 = wid * n
        iota = lax.broadcasted_iota(jnp.int32, (16,), 0)

        def sb_coords(g):
            u = u0 + g
            s = u // SB_PER_S
            nbk = (u % SB_PER_S) * G
            return s // 8, s % 8, s, nbk

        def idx_copy(g, b):
            sr, si, _, nbk = sb_coords(g)
            return pltpu.make_async_copy(
                idxp_hbm.at[sr, pl.ds(nbk, G), si], idxv[b], isem[b]
            )

        def gather_copy(g, b, kk):
            return pltpu.make_async_copy(
                table_hbm.at[idxv[b].at[kk]],
                rowsv[b].at[pl.ds(kk * 128, 128)],
                gsem[b],
            )

        def store_copy(g, b, dt):
            _, _, s, nbk = sb_coords(g)
            return pltpu.make_async_copy(
                mv[b].at[dt], out_hbm.at[s, dt, pl.ds(nbk, G)], ssem[b]
            )

        def transpose(b):
            rows = rowsv[b]
            m = mv[b]
            for kk in range(G):

                def hbody(h, rvec):
                    for d in range(EMB_DIM):
                        cols = jnp.full((16,), d, jnp.int32)
                        vals = plsc.load_gather(rows, [rvec, cols])
                        m[d // 8, kk, d % 8, pl.ds(h * 16, 16)] = vals
                    return rvec + 16

                lax.fori_loop(0, 8, hbody, iota + kk * 128)

        idx_copy(0, 0).start()

        def body(j, carry):
            for p in (0, 1):
                g = 2 * j + p
                b = p
                bo = 1 - p

                @pl.when(g < n)
                def _():
                    idx_copy(g, b).wait()
                    for kk in range(G):
                        gather_copy(g, b, kk).start()

                @pl.when(jnp.logical_and(g >= 1, g <= n))
                def _():
                    for kk in range(G):
                        gather_copy(g - 1, bo, kk).wait()

                    @pl.when(g >= 3)
                    def _():
                        for dt in range(4):
                            store_copy(g - 3, bo, dt).wait()

                    transpose(bo)
                    for dt in range(4):
                        store_copy(g - 1, bo, dt).start()

                @pl.when(g + 1 < n)
                def _():
                    idx_copy(g + 1, bo).start()

            return carry

        lax.fori_loop(0, (n + 3) // 2, body, 0)

        for dt in range(4):
            store_copy(n - 2, (n - 2) % 2, dt).wait()
        for dt in range(4):
            store_copy(n - 1, (n - 1) % 2, dt).wait()

    return gather_kernel


@jax.jit
def kernel(input, weight):
    n_seq, seq_len = input.shape
    idx = input.astype(jnp.int32)
    idxp = (
        idx.T.reshape(seq_len // 8, 8, n_seq // 128, 128).transpose(0, 2, 1, 3)
    )
    out5 = _make_gather(n_seq, seq_len)(idxp, weight)
    return out5.transpose(2, 4, 0, 1, 3).reshape(n_seq, seq_len, EMB_DIM)


# speedup vs baseline: 3.8422x; 3.8422x over previous
import functools

import jax
import jax.numpy as jnp
from jax import lax
from jax.experimental import pallas as pl
from jax.experimental.pallas import tpu as pltpu
from jax.experimental.pallas import tpu_sc as plsc

EMB_DIM = 32
NUM_CORES = 2
NUM_SUBCORES = 16
G = 4
TOK = G * 128
SB_PER_S = 128 // G


@functools.lru_cache(maxsize=None)
def _make_gather(n_seq, seq_len):
    NW = NUM_CORES * NUM_SUBCORES
    B = n_seq * seq_len
    assert B % (NW * TOK) == 0
    n = B // (NW * TOK)
    n_tiles = n_seq // 128
    s_tiles = seq_len // 8
    mesh = plsc.VectorSubcoreMesh(core_axis_name="c", subcore_axis_name="s")

    scratch = (
        [pltpu.VMEM((G, 128), jnp.int32)] * 2
        + [pltpu.VMEM((TOK, EMB_DIM), jnp.float32)] * 2
        + [pltpu.VMEM((4, G, 8, 128), jnp.float32)] * 2
        + [pltpu.SemaphoreType.DMA] * 6
    )

    @functools.partial(
        pl.kernel,
        out_type=jax.ShapeDtypeStruct(
            (seq_len, 4, n_tiles, 8, 128), jnp.float32
        ),
        mesh=mesh,
        scratch_types=scratch,
        compiler_params=pltpu.CompilerParams(
            use_tc_tiling_on_sc=False, needs_layout_passes=False
        ),
    )
    def gather_kernel(idxp_hbm, table_hbm, out_hbm, *scr):
        idxv = scr[0:2]
        rowsv = scr[2:4]
        mv = scr[4:6]
        isem = scr[6:8]
        gsem = scr[8:10]
        ssem = scr[10:12]

        wid = lax.axis_index("s") * NUM_CORES + lax.axis_index("c")
        u0 = wid * n
        iota = lax.broadcasted_iota(jnp.int32, (16,), 0)

        def sb_coords(g):
            u = u0 + g
            s = u // SB_PER_S
            nbk = (u % SB_PER_S) * G
            return s // 8, s % 8, s, nbk

        def idx_copy(g, b):
            sr, si, _, nbk = sb_coords(g)
            return pltpu.make_async_copy(
                idxp_hbm.at[sr, pl.ds(nbk, G), si], idxv[b], isem[b]
            )

        def gather_copy(g, b, kk):
            return pltpu.make_async_copy(
                table_hbm.at[idxv[b].at[kk]],
                rowsv[b].at[pl.ds(kk * 128, 128)],
                gsem[b],
            )

        def store_copy(g, b, dt):
            _, _, s, nbk = sb_coords(g)
            return pltpu.make_async_copy(
                mv[b].at[dt], out_hbm.at[s, dt, pl.ds(nbk, G)], ssem[b]
            )

        def transpose(b):
            rows = rowsv[b]
            m = mv[b]
            for kk in range(G):

                @functools.partial(plsc.parallel_loop, 0, 8, unroll=2)
                def _(h):
                    rvec = iota + (kk * 128 + h * 16)
                    for d0 in (0, 16):
                        vals = [
                            plsc.load_gather(
                                rows,
                                [rvec, jnp.full((16,), d0 + i, jnp.int32)],
                            )
                            for i in range(16)
                        ]
                        for i in range(16):
                            d = d0 + i
                            m[d // 8, kk, d % 8, pl.ds(h * 16, 16)] = vals[i]

        idx_copy(0, 0).start()

        def body(j, carry):
            for p in (0, 1):
                g = 2 * j + p
                b = p
                bo = 1 - p

                @pl.when(g < n)
                def _():
                    idx_copy(g, b).wait()
                    for kk in range(G):
                        gather_copy(g, b, kk).start()

                @pl.when(jnp.logical_and(g >= 1, g <= n))
                def _():
                    for kk in range(G):
                        gather_copy(g - 1, bo, kk).wait()

                    @pl.when(g >= 3)
                    def _():
                        for dt in range(4):
                            store_copy(g - 3, bo, dt).wait()

                    transpose(bo)
                    for dt in range(4):
                        store_copy(g - 1, bo, dt).start()

                @pl.when(g + 1 < n)
                def _():
                    idx_copy(g + 1, bo).start()

            return carry

        lax.fori_loop(0, (n + 3) // 2, body, 0)

        for dt in range(4):
            store_copy(n - 2, (n - 2) % 2, dt).wait()
        for dt in range(4):
            store_copy(n - 1, (n - 1) % 2, dt).wait()

    return gather_kernel


@jax.jit
def kernel(input, weight):
    n_seq, seq_len = input.shape
    idx = input.astype(jnp.int32)
    idxp = (
        idx.T.reshape(seq_len // 8, 8, n_seq // 128, 128).transpose(0, 2, 1, 3)
    )
    out5 = _make_gather(n_seq, seq_len)(idxp, weight)
    return out5.transpose(2, 4, 0, 1, 3).reshape(n_seq, seq_len, EMB_DIM)
